# baseline (speedup 1.0000x reference)
import functools

import jax
import jax.numpy as jnp
from jax import lax
from jax.experimental import pallas as pl
from jax.experimental.pallas import tpu as pltpu
from jax.experimental.pallas import tpu_sc as plsc

N = 16384
K = 8192
C = 256

NB = 512
KB = 2048

WINDOWS = ((0, 2736), (2736, 5472), (5472, 8192))
BIG = 2**30


def _tc_body(a_ref, fx_ref, b_ref, w_ref, g_ref, idx_ref, wv_ref, wi_ref):
    k = pl.program_id(0)
    n = pl.program_id(1)
    m = lax.dot_general(fx_ref[...], w_ref[...], (((1,), (1,)), ((), ())),
                        precision=None, preferred_element_type=jnp.float32)
    d = (a_ref[...] + b_ref[...]) - 2.0 * m
    g = g_ref[...]
    sl = pl.ds(n * NB, NB)

    def part(dm):
        pmin = jnp.min(dm, axis=1, keepdims=True)
        pidx = jnp.min(jnp.where(dm == pmin, g, BIG), axis=1, keepdims=True)
        return pmin, pidx

    def store(j, pmin, pidx):
        wv_ref[sl, j:j + 1] = pmin
        wi_ref[sl, j:j + 1] = pidx

    def merge(j, pmin, pidx):
        pv = wv_ref[sl, j:j + 1]
        pi = wi_ref[sl, j:j + 1]
        take = pmin < pv
        wv_ref[sl, j:j + 1] = jnp.where(take, pmin, pv)
        wi_ref[sl, j:j + 1] = jnp.where(take, pidx, pi)

    for kk in range(K // KB):
        @pl.when(k == kk)
        def _(kk=kk):
            lo, hi = kk * KB, (kk + 1) * KB
            for j, (s, e) in enumerate(WINDOWS):
                if s >= hi or e <= lo:
                    continue
                if s <= lo and e >= hi:
                    dm = d
                else:
                    inside = (g >= s) if s > lo else (g < e)
                    dm = jnp.where(inside, d, jnp.inf)
                pmin, pidx = part(dm)
                if s >= lo:
                    store(j, pmin, pidx)
                else:
                    merge(j, pmin, pidx)

    @pl.when(k == K // KB - 1)
    def _():
        acc_v = jnp.full((NB, 1), jnp.inf, jnp.float32)
        acc_i = jnp.zeros((NB, 1), jnp.int32)
        for j in range(len(WINDOWS)):
            acc_up = acc_v.astype(jnp.bfloat16).astype(jnp.float32)
            nv = wv_ref[sl, j:j + 1]
            ni = wi_ref[sl, j:j + 1]
            take = (nv < acc_up) | ((nv == acc_up) & (ni < acc_i))
            acc_v = jnp.where(take, nv, acc_up)
            acc_i = jnp.where(take, ni, acc_i)
        idx_ref[sl, :] = acc_i


def _make_tc(interpret=False):
    return pl.pallas_call(
        _tc_body,
        grid=(K // KB, N // NB),
        in_specs=[
            pl.BlockSpec((NB, 1), lambda k, n: (n, 0)),
            pl.BlockSpec((NB, C), lambda k, n: (n, 0)),
            pl.BlockSpec((1, KB), lambda k, n: (0, k)),
            pl.BlockSpec((KB, C), lambda k, n: (k, 0)),
            pl.BlockSpec((1, KB), lambda k, n: (0, k)),
        ],
        out_specs=pl.BlockSpec((N, 1), lambda k, n: (0, 0)),
        out_shape=jax.ShapeDtypeStruct((N, 1), jnp.int32),
        scratch_shapes=[
            pltpu.VMEM((N, 8), jnp.float32),
            pltpu.VMEM((N, 8), jnp.int32),
        ],
        interpret=interpret,
    )


_CH = 256


def _make_sc_gather():
    info = plsc.get_sparse_core_info()
    nc, ns = info.num_cores, info.num_subcores
    nw = nc * ns
    b_per_w = N // nw
    chunks = b_per_w // _CH
    mesh = plsc.VectorSubcoreMesh(core_axis_name="c", subcore_axis_name="s")

    @functools.partial(
        pl.kernel, mesh=mesh,
        out_type=jax.ShapeDtypeStruct((N, C), jnp.float32),
        scratch_types=[
            pltpu.VMEM((_CH,), jnp.int32),
            pltpu.VMEM((_CH, C), jnp.float32),
            pltpu.SemaphoreType.DMA,
        ],
    )
    def gather_k(idx_hbm, table_hbm, out_hbm, idx_v, rows_v, sem):
        wid = lax.axis_index("s") * nc + lax.axis_index("c")
        for ci in range(chunks):
            base = wid * b_per_w + ci * _CH
            pltpu.sync_copy(idx_hbm.at[pl.ds(base, _CH)], idx_v)
            pltpu.async_copy(table_hbm.at[idx_v], rows_v, sem).wait()
            pltpu.sync_copy(rows_v, out_hbm.at[pl.ds(base, _CH)])

    return gather_k


def kernel(x, weight):
    flat_x = jnp.transpose(x, (0, 2, 3, 4, 1)).reshape(-1, C)
    a = jnp.sum(flat_x ** 2, axis=1, keepdims=True)
    b = jnp.sum(weight ** 2, axis=1).reshape(1, K)
    g = jnp.arange(K, dtype=jnp.int32).reshape(1, K)
    idx = _make_tc()(a, flat_x, b, weight, g)
    q = _make_sc_gather()(idx.reshape(N), weight)
    return q.reshape(x.shape)


# speedup vs baseline: 1.1181x; 1.1181x over previous
import functools

import jax
import jax.numpy as jnp
from jax import lax
from jax.experimental import pallas as pl
from jax.experimental.pallas import tpu as pltpu
from jax.experimental.pallas import tpu_sc as plsc

N = 16384
K = 8192
C = 256

NB = 512
KB = 2048

WINDOWS = ((0, 2736), (2736, 5472), (5472, 8192))
BIG = 2**30


def _tc_body(a_ref, fx_ref, b_ref, w_ref, g_ref, idx_ref, wv_ref, wi_ref):
    k = pl.program_id(0)
    n = pl.program_id(1)
    m = lax.dot_general(fx_ref[...], w_ref[...], (((1,), (1,)), ((), ())),
                        precision=None, preferred_element_type=jnp.float32)
    d = (a_ref[...] + b_ref[...]) - 2.0 * m
    g = g_ref[...]
    sl = pl.ds(n * NB, NB)

    def part(dm):
        pmin = jnp.min(dm, axis=1, keepdims=True)
        pidx = jnp.min(jnp.where(dm == pmin, g, jnp.float32(BIG)),
                       axis=1, keepdims=True)
        return pmin, pidx

    def store(j, pmin, pidx):
        wv_ref[sl, j:j + 1] = pmin
        wi_ref[sl, j:j + 1] = pidx

    def merge(j, pmin, pidx):
        pv = wv_ref[sl, j:j + 1]
        pi = wi_ref[sl, j:j + 1]
        take = pmin < pv
        wv_ref[sl, j:j + 1] = jnp.where(take, pmin, pv)
        wi_ref[sl, j:j + 1] = jnp.where(take, pidx, pi)

    for kk in range(K // KB):
        @pl.when(k == kk)
        def _(kk=kk):
            lo, hi = kk * KB, (kk + 1) * KB
            for j, (s, e) in enumerate(WINDOWS):
                if s >= hi or e <= lo:
                    continue
                if s <= lo and e >= hi:
                    dm = d
                else:
                    inside = (g >= s) if s > lo else (g < e)
                    dm = jnp.where(inside, d, jnp.inf)
                pmin, pidx = part(dm)
                if s >= lo:
                    store(j, pmin, pidx)
                else:
                    merge(j, pmin, pidx)

    @pl.when(k == K // KB - 1)
    def _():
        acc_v = jnp.full((NB, 1), jnp.inf, jnp.float32)
        acc_i = jnp.zeros((NB, 1), jnp.float32)
        for j in range(len(WINDOWS)):
            acc_up = acc_v.astype(jnp.bfloat16).astype(jnp.float32)
            nv = wv_ref[sl, j:j + 1]
            ni = wi_ref[sl, j:j + 1]
            take = (nv < acc_up) | ((nv == acc_up) & (ni < acc_i))
            acc_v = jnp.where(take, nv, acc_up)
            acc_i = jnp.where(take, ni, acc_i)
        idx_ref[sl, :] = acc_i.astype(jnp.int32)


def _make_tc(interpret=False):
    return pl.pallas_call(
        _tc_body,
        grid=(K // KB, N // NB),
        in_specs=[
            pl.BlockSpec((NB, 1), lambda k, n: (n, 0)),
            pl.BlockSpec((NB, C), lambda k, n: (n, 0)),
            pl.BlockSpec((1, KB), lambda k, n: (0, k)),
            pl.BlockSpec((KB, C), lambda k, n: (k, 0)),
            pl.BlockSpec((1, KB), lambda k, n: (0, k)),
        ],
        out_specs=pl.BlockSpec((N, 1), lambda k, n: (0, 0)),
        out_shape=jax.ShapeDtypeStruct((N, 1), jnp.int32),
        scratch_shapes=[
            pltpu.VMEM((N, 8), jnp.float32),
            pltpu.VMEM((N, 8), jnp.float32),
        ],
        interpret=interpret,
    )


_CH = 256


def _make_sc_gather():
    info = plsc.get_sparse_core_info()
    nc, ns = info.num_cores, info.num_subcores
    nw = nc * ns
    b_per_w = N // nw
    chunks = b_per_w // _CH
    mesh = plsc.VectorSubcoreMesh(core_axis_name="c", subcore_axis_name="s")

    @functools.partial(
        pl.kernel, mesh=mesh,
        out_type=jax.ShapeDtypeStruct((N, C), jnp.float32),
        scratch_types=[
            pltpu.VMEM((_CH,), jnp.int32),
            pltpu.VMEM((_CH, C), jnp.float32),
            pltpu.SemaphoreType.DMA,
        ],
    )
    def gather_k(idx_hbm, table_hbm, out_hbm, idx_v, rows_v, sem):
        wid = lax.axis_index("s") * nc + lax.axis_index("c")
        for ci in range(chunks):
            base = wid * b_per_w + ci * _CH
            pltpu.sync_copy(idx_hbm.at[pl.ds(base, _CH)], idx_v)
            pltpu.async_copy(table_hbm.at[idx_v], rows_v, sem).wait()
            pltpu.sync_copy(rows_v, out_hbm.at[pl.ds(base, _CH)])

    return gather_k


def kernel(x, weight):
    flat_x = jnp.transpose(x, (0, 2, 3, 4, 1)).reshape(-1, C)
    a = jnp.sum(flat_x ** 2, axis=1, keepdims=True)
    b = jnp.sum(weight ** 2, axis=1).reshape(1, K)
    g = jnp.arange(K, dtype=jnp.float32).reshape(1, K)
    idx = _make_tc()(a, flat_x, b, weight, g)
    q = _make_sc_gather()(idx.reshape(N), weight)
    return q.reshape(x.shape)


# speedup vs baseline: 1.2084x; 1.0807x over previous
import functools

import jax
import jax.numpy as jnp
from jax import lax
from jax.experimental import pallas as pl
from jax.experimental.pallas import tpu as pltpu
from jax.experimental.pallas import tpu_sc as plsc

N = 16384
K = 8192
C = 256

NB = 1024
KB = 2048

WINDOWS = ((0, 2736), (2736, 5472), (5472, 8192))
BIG = 2**30


def _tc_body(a_ref, fx_ref, b_ref, w_ref, g_ref, idx_ref, wv_ref, wi_ref):
    k = pl.program_id(0)
    n = pl.program_id(1)
    m = lax.dot_general(fx_ref[...], w_ref[...], (((1,), (1,)), ((), ())),
                        precision=None, preferred_element_type=jnp.float32)
    d = (a_ref[...] + b_ref[...]) - 2.0 * m
    g = g_ref[...]
    sl = pl.ds(n * NB, NB)

    def part(dm):
        pmin = jnp.min(dm, axis=1, keepdims=True)
        pidx = jnp.min(jnp.where(dm == pmin, g, jnp.float32(BIG)),
                       axis=1, keepdims=True)
        return pmin, pidx

    def store(j, pmin, pidx):
        wv_ref[sl, j:j + 1] = pmin
        wi_ref[sl, j:j + 1] = pidx

    def merge(j, pmin, pidx):
        pv = wv_ref[sl, j:j + 1]
        pi = wi_ref[sl, j:j + 1]
        take = pmin < pv
        wv_ref[sl, j:j + 1] = jnp.where(take, pmin, pv)
        wi_ref[sl, j:j + 1] = jnp.where(take, pidx, pi)

    for kk in range(K // KB):
        @pl.when(k == kk)
        def _(kk=kk):
            lo, hi = kk * KB, (kk + 1) * KB
            for j, (s, e) in enumerate(WINDOWS):
                if s >= hi or e <= lo:
                    continue
                if s <= lo and e >= hi:
                    dm = d
                else:
                    inside = (g >= s) if s > lo else (g < e)
                    dm = jnp.where(inside, d, jnp.inf)
                pmin, pidx = part(dm)
                if s >= lo:
                    store(j, pmin, pidx)
                else:
                    merge(j, pmin, pidx)

    @pl.when(k == K // KB - 1)
    def _():
        acc_v = jnp.full((NB, 1), jnp.inf, jnp.float32)
        acc_i = jnp.zeros((NB, 1), jnp.float32)
        for j in range(len(WINDOWS)):
            acc_up = acc_v.astype(jnp.bfloat16).astype(jnp.float32)
            nv = wv_ref[sl, j:j + 1]
            ni = wi_ref[sl, j:j + 1]
            take = (nv < acc_up) | ((nv == acc_up) & (ni < acc_i))
            acc_v = jnp.where(take, nv, acc_up)
            acc_i = jnp.where(take, ni, acc_i)
        idx_ref[sl, :] = acc_i.astype(jnp.int32)


def _make_tc(interpret=False):
    return pl.pallas_call(
        _tc_body,
        grid=(K // KB, N // NB),
        in_specs=[
            pl.BlockSpec((NB, 1), lambda k, n: (n, 0)),
            pl.BlockSpec((NB, C), lambda k, n: (n, 0)),
            pl.BlockSpec((1, KB), lambda k, n: (0, k)),
            pl.BlockSpec((KB, C), lambda k, n: (k, 0)),
            pl.BlockSpec((1, KB), lambda k, n: (0, k)),
        ],
        out_specs=pl.BlockSpec((N, 1), lambda k, n: (0, 0)),
        out_shape=jax.ShapeDtypeStruct((N, 1), jnp.int32),
        scratch_shapes=[
            pltpu.VMEM((N, 8), jnp.float32),
            pltpu.VMEM((N, 8), jnp.float32),
        ],
        interpret=interpret,
    )


_CH = 256


def _make_sc_gather():
    info = plsc.get_sparse_core_info()
    nc, ns = info.num_cores, info.num_subcores
    nw = nc * ns
    b_per_w = N // nw
    chunks = b_per_w // _CH
    mesh = plsc.VectorSubcoreMesh(core_axis_name="c", subcore_axis_name="s")

    @functools.partial(
        pl.kernel, mesh=mesh,
        out_type=jax.ShapeDtypeStruct((N, C), jnp.float32),
        scratch_types=[
            pltpu.VMEM((_CH,), jnp.int32),
            pltpu.VMEM((_CH, C), jnp.float32),
            pltpu.SemaphoreType.DMA,
        ],
    )
    def gather_k(idx_hbm, table_hbm, out_hbm, idx_v, rows_v, sem):
        wid = lax.axis_index("s") * nc + lax.axis_index("c")
        for ci in range(chunks):
            base = wid * b_per_w + ci * _CH
            pltpu.sync_copy(idx_hbm.at[pl.ds(base, _CH)], idx_v)
            pltpu.async_copy(table_hbm.at[idx_v], rows_v, sem).wait()
            pltpu.sync_copy(rows_v, out_hbm.at[pl.ds(base, _CH)])

    return gather_k


def kernel(x, weight):
    flat_x = jnp.transpose(x, (0, 2, 3, 4, 1)).reshape(-1, C)
    a = jnp.sum(flat_x ** 2, axis=1, keepdims=True)
    b = jnp.sum(weight ** 2, axis=1).reshape(1, K)
    g = jnp.arange(K, dtype=jnp.float32).reshape(1, K)
    idx = _make_tc()(a, flat_x, b, weight, g)
    q = _make_sc_gather()(idx.reshape(N), weight)
    return q.reshape(x.shape)
